# baseline (speedup 1.0000x reference)
import jax
import jax.numpy as jnp
from jax.experimental import pallas as pl
from jax.experimental.pallas import tpu as pltpu

_K = 68
_HW = 64 * 64


def _stage1_kernel(o11_ref, o12_ref, o21_ref, o22_ref, t1_ref, t2_ref, r_ref):
    t1 = t1_ref[0]
    t2x = t2_ref[0, 0]
    t2y = t2_ref[0, 1]

    d1 = o11_ref[0] - t1
    r1 = jnp.sum(d1 * d1, axis=1)

    px = o12_ref[0, 0] * t1 - t2x
    py = o12_ref[0, 1] * t1 - t2y
    r2 = jnp.sum(px * px + py * py, axis=1)

    d3 = o21_ref[0] - t1
    r3 = jnp.sum(d3 * d3, axis=1)

    dx = o22_ref[0, 0] - t2x
    dy = o22_ref[0, 1] - t2y
    r4 = jnp.sum((dx * dx + dy * dy) * (t1 * t1), axis=1)

    r_ref[0] = jnp.stack([r1, r2, r3, r4], axis=0)


def _stage2_kernel(r_ref, w_ref, o_ref):
    B = r_ref.shape[0]
    K = r_ref.shape[2]
    r = r_ref[...]
    r1 = r[:, 0, :]
    r2 = r[:, 1, :]
    r3 = r[:, 2, :]
    r4 = r[:, 3, :]

    v = r3 * (0.5 / B)

    vi = v[:, :, None]
    vj = v[:, None, :]
    jj = jax.lax.broadcasted_iota(jnp.int32, (B, K, K), 2)
    kk = jax.lax.broadcasted_iota(jnp.int32, (B, K, K), 1)
    beats = (vj > vi) | ((vj == vi) & (jj < kk))
    rank = jnp.sum(beats.astype(jnp.int32), axis=2)
    mask = (rank < (K // 2)).astype(jnp.float32)

    inv_n = 1.0 / (B * K * _HW)
    loss1_1 = jnp.sum(r1) * inv_n
    loss1_2 = jnp.sum(r2) * inv_n
    loss2_1_m = jnp.sum(v * mask) / (B * K)
    loss2_2 = jnp.sum(r4 * mask) * inv_n

    w = w_ref[...]
    loss = ((loss1_1 + loss2_1_m) * w[0, 0]
            + (loss1_2 + loss2_2 * 5.0) * w[0, 1])
    o_ref[...] = jnp.reshape(loss, (1, 1))


def kernel(out1_1, out1_2, out2_1, out2_2, targets1, targets2, weights):
    B, K = out1_1.shape[0], out1_1.shape[1]
    o11 = out1_1.reshape(B, K, _HW)
    o12 = out1_2.reshape(B, 2, K, _HW)
    o21 = out2_1.reshape(B, K, _HW)
    o22 = out2_2.reshape(B, 2, K, _HW)
    t1 = targets1.reshape(B, K, _HW)
    t2 = targets2.reshape(B, 2, K, _HW)

    spec3 = pl.BlockSpec((1, K, _HW), lambda i: (i, 0, 0))
    spec4 = pl.BlockSpec((1, 2, K, _HW), lambda i: (i, 0, 0, 0))

    partials = pl.pallas_call(
        _stage1_kernel,
        grid=(B,),
        in_specs=[spec3, spec4, spec3, spec4, spec3, spec4],
        out_specs=pl.BlockSpec((1, 4, K), lambda i: (i, 0, 0)),
        out_shape=jax.ShapeDtypeStruct((B, 4, K), jnp.float32),
        compiler_params=pltpu.CompilerParams(
            dimension_semantics=("arbitrary",),
        ),
    )(o11, o12, o21, o22, t1, t2)

    loss = pl.pallas_call(
        _stage2_kernel,
        in_specs=[
            pl.BlockSpec((B, 4, K), lambda: (0, 0, 0)),
            pl.BlockSpec((1, 2), lambda: (0, 0)),
        ],
        out_specs=pl.BlockSpec((1, 1), lambda: (0, 0)),
        out_shape=jax.ShapeDtypeStruct((1, 1), jnp.float32),
    )(partials, weights.reshape(1, 2))

    return loss[0, 0]
